# baseline (speedup 1.0000x reference)
import jax
import jax.numpy as jnp
from jax.experimental import pallas as pl

NUM_CLASSES = 1000


def _body(xt_ref, o_ref):
    cb = o_ref.shape[1]
    cls = jax.lax.broadcasted_iota(jnp.int32, o_ref.shape, 1) + pl.program_id(1) * cb
    o_ref[...] = (cls == xt_ref[...]).astype(jnp.float32)


def kernel(x):
    B, L = x.shape
    xt = jnp.swapaxes(x, 0, 1).reshape(L, 1, B)
    CB = 200
    p = pl.pallas_call(
        _body,
        grid=(L, NUM_CLASSES // CB),
        in_specs=[pl.BlockSpec((1, 1, B), lambda i, j: (i, 0, 0))],
        out_specs=pl.BlockSpec((1, CB, B), lambda i, j: (i, j, 0)),
        out_shape=jax.ShapeDtypeStruct((L, NUM_CLASSES, B), jnp.float32),
    )(xt)
    return jnp.transpose(p, (2, 1, 0))


# speedup vs baseline: 2.0581x; 2.0581x over previous
import jax
import jax.numpy as jnp
from jax.experimental import pallas as pl
from jax.experimental.pallas import tpu as pltpu

NUM_CLASSES = 1000


def _body(x_ref, o_ref, xt_ref):
    l = pl.program_id(0)

    @pl.when(l == 0)
    def _():
        xt_ref[pl.ds(0, 50), :] = jnp.swapaxes(x_ref[...], 0, 1)

    xcol = xt_ref[pl.ds(l, 1), :].reshape(1, 1, x_ref.shape[0])
    cls = jax.lax.broadcasted_iota(jnp.int32, o_ref.shape, 1)
    o_ref[...] = (cls == xcol).astype(jnp.float32)


def kernel(x):
    B, L = x.shape
    p = pl.pallas_call(
        _body,
        grid=(L,),
        in_specs=[pl.BlockSpec((B, L), lambda i: (0, 0))],
        out_specs=pl.BlockSpec((1, NUM_CLASSES, B), lambda i: (i, 0, 0)),
        out_shape=jax.ShapeDtypeStruct((L, NUM_CLASSES, B), jnp.float32),
        scratch_shapes=[pltpu.VMEM((56, B), jnp.int32)],
    )(x)
    return jnp.transpose(p, (2, 1, 0))


# speedup vs baseline: 2.0788x; 1.0101x over previous
import jax
import jax.numpy as jnp
from jax.experimental import pallas as pl

NUM_CLASSES = 1000


def _body(xt_ref, o_ref):
    cls = jax.lax.broadcasted_iota(jnp.int32, o_ref.shape, 1)
    o_ref[...] = (cls == xt_ref[...]).astype(jnp.float32)


def kernel(x):
    B, L = x.shape
    xt = jnp.swapaxes(x, 0, 1).reshape(L, 1, B)
    p = pl.pallas_call(
        _body,
        grid=(L,),
        in_specs=[pl.BlockSpec((1, 1, B), lambda i: (i, 0, 0))],
        out_specs=pl.BlockSpec((1, NUM_CLASSES, B), lambda i: (i, 0, 0)),
        out_shape=jax.ShapeDtypeStruct((L, NUM_CLASSES, B), jnp.float32),
    )(xt)
    return jnp.transpose(p, (2, 1, 0))
